# Initial kernel scaffold; baseline (speedup 1.0000x reference)
import jax
import jax.numpy as jnp
from jax.experimental import pallas as pl


def kernel(feature_tensor, week_tensor, stamptensor, a0, a1, a2, k, params):
    raise NotImplementedError("write your pallas kernel here")


# speedup vs baseline: 1.2259x; 1.2259x over previous
import math

import jax
import jax.numpy as jnp
import numpy as np
from jax.experimental import pallas as pl

_S, _N = 8, 97


def _pe8_np():
    pos = np.arange(20, dtype=np.float32)[:, None]
    div = np.exp(np.arange(0, 16, 2, dtype=np.float32) * (-math.log(10000.0) / 16.0))
    pe = np.zeros((20, 16), dtype=np.float32)
    pe[:, 0::2] = np.sin(pos * div)
    pe[:, 1::2] = np.cos(pos * div)
    return pe[:_S]


_PE8 = _pe8_np()


def _mm_t(x, w):
    return jax.lax.dot_general(
        x, w, (((1,), (1,)), ((), ())), preferred_element_type=jnp.float32
    )


def _mm(x, w):
    return jax.lax.dot_general(
        x, w, (((1,), (0,)), ((), ())), preferred_element_type=jnp.float32
    )


def _ln(x, g, b, eps):
    m = jnp.mean(x, axis=-1, keepdims=True)
    v = jnp.mean((x - m) * (x - m), axis=-1, keepdims=True)
    return (x - m) * jax.lax.rsqrt(v + eps) * g + b


def _sel(rows, cols, shift):
    r = jax.lax.broadcasted_iota(jnp.int32, (rows, cols), 0)
    c = jax.lax.broadcasted_iota(jnp.int32, (rows, cols), 1)
    return (c == r + shift).astype(jnp.float32)


def _fused_body(
    feat, week, stamp, a0, a1, a2, pe,
    emb1, emb2, lin0_w, lin1_w, lin2_w, lin0_b, lin1_b, lin2_b,
    gc10_w, gc10_b, gc11_w, gc11_b,
    gc20_w, gc20_b, gc21_w, gc21_b,
    gc30_w, gc30_b, gc31_w, gc31_b,
    fw0, fw1, fw2, gcn_g, gcn_b,
    attn_in_w, attn_in_b, attn_out_w, attn_out_b,
    n1_g, n1_b, ff1_w, ff1_b, ff2_w, ff2_b, n2_g, n2_b, en_g, en_b,
    pred_w, pred_b, out0_w, out0_b, out1_w, out1_b,
    r1_ref, r2_ref,
):
    f32 = jnp.float32
    A0, A1, A2 = a0[:, :], a1[:, :], a2[:, :]

    w3f = _mm(jnp.transpose(lin2_w[:, :]), _sel(8, 16, 0))
    w2f = _mm(_mm_t(emb2[:, :], lin1_w[:, :]), _sel(4, 16, 8))
    w1f = _mm(_mm_t(emb1[:, :], lin0_w[:, :]), _sel(4, 16, 12))
    bias16 = (
        _mm(lin2_b[:, :], _sel(8, 16, 0))
        + _mm(lin1_b[:, :], _sel(4, 16, 8))
        + _mm(lin0_b[:, :], _sel(4, 16, 12))
    )

    def gcn_branch(x, A, w0, b0, w1, b1):
        u = _mm(x, w0[:, :])
        h = jnp.maximum(_mm(A, u) + b0[:, :], 0.0)
        z = _mm(A, _mm(h, w1[:, :])) + b1[:, :]
        return z

    qs, ks, vs, srcs = [], [], [], []
    wq = attn_in_w[0:16, :]
    wk = attn_in_w[16:32, :]
    wv = attn_in_w[32:48, :]
    bq = _mm(attn_in_b[:, :], _sel(16, 48, 0).T)
    bk = _mm(attn_in_b[:, :], _sel(16, 48, 16).T)
    bv = _mm(attn_in_b[:, :], _sel(16, 48, 32).T)

    for i in range(_S):
        f_i = feat[i]
        wk_i = week[i]
        st_i = stamp[i]
        oh_w = (wk_i == jax.lax.broadcasted_iota(jnp.int32, (_N, 8), 1)).astype(f32)
        oh_s = (st_i == jax.lax.broadcasted_iota(jnp.int32, (_N, 5), 1)).astype(f32)
        x = _mm(f_i, w3f) + _mm(oh_s, w2f) + _mm(oh_w, w1f) + bias16

        z0 = gcn_branch(x, A0, gc10_w, gc10_b, gc11_w, gc11_b)
        z1 = gcn_branch(x, A1, gc20_w, gc20_b, gc21_w, gc21_b)
        z2 = gcn_branch(x, A2, gc30_w, gc30_b, gc31_w, gc31_b)
        xo = _mm(z0, fw0[:, :]) + _mm(z1, fw1[:, :]) + _mm(z2, fw2[:, :])
        xg = _ln(xo + x, gcn_g[:, :], gcn_b[:, :], 1e-6)

        src = xg + pe[i : i + 1, :]
        srcs.append(src)
        qs.append(_mm_t(src, wq) + bq)
        ks.append(_mm_t(src, wk) + bk)
        vs.append(_mm_t(src, wv) + bv)

    G = (
        jax.lax.broadcasted_iota(jnp.int32, (16, 4), 0) // 4
        == jax.lax.broadcasted_iota(jnp.int32, (16, 4), 1)
    ).astype(f32)

    for i in range(_S):
        scores = [_mm(qs[i] * ks[j], G) * 0.5 for j in range(_S)]
        m = scores[0]
        for j in range(1, _S):
            m = jnp.maximum(m, scores[j])
        exps = [jnp.exp(s - m) for s in scores]
        den = exps[0]
        for j in range(1, _S):
            den = den + exps[j]
        inv = 1.0 / den
        ao = jnp.zeros((_N, 16), f32)
        for j in range(_S):
            ao = ao + _mm_t(exps[j] * inv, G) * vs[j]

        ao = _mm_t(ao, attn_out_w[:, :]) + attn_out_b[:, :]
        x1 = _ln(srcs[i] + ao, n1_g[:, :], n1_b[:, :], 1e-5)
        h = jnp.maximum(_mm_t(x1, ff1_w[:, :]) + ff1_b[:, :], 0.0)
        y = _mm_t(h, ff2_w[:, :]) + ff2_b[:, :]
        x2 = _ln(x1 + y, n2_g[:, :], n2_b[:, :], 1e-5)
        enc = _ln(x2, en_g[:, :], en_b[:, :], 1e-6)

        r1 = _mm_t(enc, pred_w[:, :]) + pred_b[:, :]
        rb = _mm_t(r1, out0_w[:, :]) + out0_b[:, :]
        r2 = jnp.sum(rb * out1_w[:, :], axis=-1, keepdims=True) + out1_b[0, 0]
        r1_ref[i] = r1
        r2_ref[i] = r2


def kernel(feature_tensor, week_tensor, stamptensor, a0, a1, a2, k, params):
    p = params
    del k
    feat = feature_tensor.reshape(_S, _N, 1)
    week = week_tensor.reshape(_S, _N, 1)
    stamp = stamptensor.reshape(_S, _N, 1)
    pe = jnp.asarray(_PE8)
    args = [
        feat, week, stamp, a0, a1, a2, pe,
        p["emb1"], p["emb2"], p["lin0_w"], p["lin1_w"], p["lin2_w"],
        p["lin0_b"].reshape(1, 4), p["lin1_b"].reshape(1, 4), p["lin2_b"].reshape(1, 8),
        p["gc10_w"], p["gc10_b"].reshape(1, 64), p["gc11_w"], p["gc11_b"].reshape(1, 32),
        p["gc20_w"], p["gc20_b"].reshape(1, 64), p["gc21_w"], p["gc21_b"].reshape(1, 32),
        p["gc30_w"], p["gc30_b"].reshape(1, 64), p["gc31_w"], p["gc31_b"].reshape(1, 32),
        p["fw0"], p["fw1"], p["fw2"],
        p["gcn_ln_g"].reshape(1, 16), p["gcn_ln_b"].reshape(1, 16),
        p["attn_in_w"], p["attn_in_b"].reshape(1, 48),
        p["attn_out_w"], p["attn_out_b"].reshape(1, 16),
        p["norm1_g"].reshape(1, 16), p["norm1_b"].reshape(1, 16),
        p["ff1_w"], p["ff1_b"].reshape(1, 2048),
        p["ff2_w"], p["ff2_b"].reshape(1, 16),
        p["norm2_g"].reshape(1, 16), p["norm2_b"].reshape(1, 16),
        p["enc_norm_g"].reshape(1, 16), p["enc_norm_b"].reshape(1, 16),
        p["pred_w"], p["pred_b"].reshape(1, 8),
        p["out0_w"], p["out0_b"].reshape(1, 4),
        p["out1_w"], p["out1_b"].reshape(1, 1),
    ]
    r1, r2 = pl.pallas_call(
        _fused_body,
        out_shape=[
            jax.ShapeDtypeStruct((_S, _N, 8), jnp.float32),
            jax.ShapeDtypeStruct((_S, _N, 1), jnp.float32),
        ],
    )(*args)
    return (r1, r2, r2[-1])


# speedup vs baseline: 1.8001x; 1.4685x over previous
import math

import jax
import jax.numpy as jnp
import numpy as np
from jax.experimental import pallas as pl

_S, _N, _NP = 8, 97, 104
_R = _S * _NP


def _pe8_np():
    pos = np.arange(20, dtype=np.float32)[:, None]
    div = np.exp(np.arange(0, 16, 2, dtype=np.float32) * (-math.log(10000.0) / 16.0))
    pe = np.zeros((20, 16), dtype=np.float32)
    pe[:, 0::2] = np.sin(pos * div)
    pe[:, 1::2] = np.cos(pos * div)
    return pe[:_S]


_PE8 = _pe8_np()


def _mm_t(x, w):
    return jax.lax.dot_general(
        x, w, (((1,), (1,)), ((), ())), preferred_element_type=jnp.float32
    )


def _mm(x, w):
    return jax.lax.dot_general(
        x, w, (((1,), (0,)), ((), ())), preferred_element_type=jnp.float32
    )


def _ln(x, g, b, eps):
    m = jnp.mean(x, axis=-1, keepdims=True)
    v = jnp.mean((x - m) * (x - m), axis=-1, keepdims=True)
    return (x - m) * jax.lax.rsqrt(v + eps) * g + b


def _sel(rows, cols, shift):
    r = jax.lax.broadcasted_iota(jnp.int32, (rows, cols), 0)
    c = jax.lax.broadcasted_iota(jnp.int32, (rows, cols), 1)
    return (c == r + shift).astype(jnp.float32)


def _blk(x, i):
    return x[i * _NP : (i + 1) * _NP, :]


def _fused_body(
    feat, week, stamp, a0, a1, a2, pe,
    emb1, emb2, lin0_w, lin1_w, lin2_w, lin0_b, lin1_b, lin2_b,
    gc10_w, gc10_b, gc11_w, gc11_b,
    gc20_w, gc20_b, gc21_w, gc21_b,
    gc30_w, gc30_b, gc31_w, gc31_b,
    fw0, fw1, fw2, gcn_g, gcn_b,
    attn_in_w, attn_in_b, attn_out_w, attn_out_b,
    n1_g, n1_b, ff1_w, ff1_b, ff2_w, ff2_b, n2_g, n2_b, en_g, en_b,
    pred_w, pred_b, out0_w, out0_b, out1_w, out1_b,
    r1_ref, r2_ref,
):
    f32 = jnp.float32
    A0, A1, A2 = a0[:, :], a1[:, :], a2[:, :]

    w3f = _mm(jnp.transpose(lin2_w[:, :]), _sel(8, 16, 0))
    w2f = _mm(_mm_t(emb2[:, :], lin1_w[:, :]), _sel(4, 16, 8))
    w1f = _mm(_mm_t(emb1[:, :], lin0_w[:, :]), _sel(4, 16, 12))
    bias16 = (
        _mm(lin2_b[:, :], _sel(8, 16, 0))
        + _mm(lin1_b[:, :], _sel(4, 16, 8))
        + _mm(lin0_b[:, :], _sel(4, 16, 12))
    )

    oh_w = (week[:, :] == jax.lax.broadcasted_iota(jnp.int32, (_R, 8), 1)).astype(f32)
    oh_s = (stamp[:, :] == jax.lax.broadcasted_iota(jnp.int32, (_R, 5), 1)).astype(f32)
    X = _mm(feat[:, :], w3f) + _mm(oh_s, w2f) + _mm(oh_w, w1f) + bias16

    def gcn_branch(A, w0, b0, w1, b1):
        U = _mm(X, w0[:, :])
        V = jnp.concatenate([_mm(A, _blk(U, i)) for i in range(_S)], axis=0)
        H = jnp.maximum(V + b0[:, :], 0.0)
        W = _mm(H, w1[:, :])
        Z = jnp.concatenate([_mm(A, _blk(W, i)) for i in range(_S)], axis=0)
        return Z + b1[:, :]

    z0 = gcn_branch(A0, gc10_w, gc10_b, gc11_w, gc11_b)
    z1 = gcn_branch(A1, gc20_w, gc20_b, gc21_w, gc21_b)
    z2 = gcn_branch(A2, gc30_w, gc30_b, gc31_w, gc31_b)
    xo = _mm(z0, fw0[:, :]) + _mm(z1, fw1[:, :]) + _mm(z2, fw2[:, :])
    xg = _ln(xo + X, gcn_g[:, :], gcn_b[:, :], 1e-6)

    step_oh = (
        jax.lax.broadcasted_iota(jnp.int32, (_R, 8), 0) // _NP
        == jax.lax.broadcasted_iota(jnp.int32, (_R, 8), 1)
    ).astype(f32)
    src = xg + _mm(step_oh, pe[:, :])

    wq = attn_in_w[0:16, :]
    wk = attn_in_w[16:32, :]
    wv = attn_in_w[32:48, :]
    bq = _mm(attn_in_b[:, :], _sel(16, 48, 0).T)
    bk = _mm(attn_in_b[:, :], _sel(16, 48, 16).T)
    bv = _mm(attn_in_b[:, :], _sel(16, 48, 32).T)
    q_all = _mm_t(src, wq) + bq
    k_all = _mm_t(src, wk) + bk
    v_all = _mm_t(src, wv) + bv

    G = (
        jax.lax.broadcasted_iota(jnp.int32, (16, 4), 0) // 4
        == jax.lax.broadcasted_iota(jnp.int32, (16, 4), 1)
    ).astype(f32)

    def tile_steps(x):
        return jnp.concatenate([x] * _S, axis=0)

    scores = []
    for j in range(_S):
        kt = tile_steps(_blk(k_all, j))
        scores.append(_mm(q_all * kt, G) * 0.5)
    m = scores[0]
    for j in range(1, _S):
        m = jnp.maximum(m, scores[j])
    exps = [jnp.exp(s - m) for s in scores]
    den = exps[0]
    for j in range(1, _S):
        den = den + exps[j]
    inv = 1.0 / den
    ao = jnp.zeros((_R, 16), f32)
    for j in range(_S):
        vt = tile_steps(_blk(v_all, j))
        ao = ao + _mm_t(exps[j] * inv, G) * vt

    ao = _mm_t(ao, attn_out_w[:, :]) + attn_out_b[:, :]
    x1 = _ln(src + ao, n1_g[:, :], n1_b[:, :], 1e-5)
    h = jnp.maximum(_mm_t(x1, ff1_w[:, :]) + ff1_b[:, :], 0.0)
    y = _mm_t(h, ff2_w[:, :]) + ff2_b[:, :]
    x2 = _ln(x1 + y, n2_g[:, :], n2_b[:, :], 1e-5)
    enc = _ln(x2, en_g[:, :], en_b[:, :], 1e-6)

    r1 = _mm_t(enc, pred_w[:, :]) + pred_b[:, :]
    rb = _mm_t(r1, out0_w[:, :]) + out0_b[:, :]
    r2 = jnp.sum(rb * out1_w[:, :], axis=-1, keepdims=True) + out1_b[0, 0]
    r1_ref[:, :] = r1
    r2_ref[:, :] = r2


def _pad_rows(x):
    return jnp.pad(x, ((0, 0), (0, _NP - _N)))


def kernel(feature_tensor, week_tensor, stamptensor, a0, a1, a2, k, params):
    p = params
    del k
    feat = _pad_rows(feature_tensor).reshape(_R, 1)
    week = _pad_rows(week_tensor).reshape(_R, 1)
    stamp = _pad_rows(stamptensor).reshape(_R, 1)
    ap = jnp.zeros((_NP, _NP), jnp.float32)
    a0p = ap.at[:_N, :_N].set(a0)
    a1p = ap.at[:_N, :_N].set(a1)
    a2p = ap.at[:_N, :_N].set(a2)
    pe = jnp.asarray(_PE8)
    args = [
        feat, week, stamp, a0p, a1p, a2p, pe,
        p["emb1"], p["emb2"], p["lin0_w"], p["lin1_w"], p["lin2_w"],
        p["lin0_b"].reshape(1, 4), p["lin1_b"].reshape(1, 4), p["lin2_b"].reshape(1, 8),
        p["gc10_w"], p["gc10_b"].reshape(1, 64), p["gc11_w"], p["gc11_b"].reshape(1, 32),
        p["gc20_w"], p["gc20_b"].reshape(1, 64), p["gc21_w"], p["gc21_b"].reshape(1, 32),
        p["gc30_w"], p["gc30_b"].reshape(1, 64), p["gc31_w"], p["gc31_b"].reshape(1, 32),
        p["fw0"], p["fw1"], p["fw2"],
        p["gcn_ln_g"].reshape(1, 16), p["gcn_ln_b"].reshape(1, 16),
        p["attn_in_w"], p["attn_in_b"].reshape(1, 48),
        p["attn_out_w"], p["attn_out_b"].reshape(1, 16),
        p["norm1_g"].reshape(1, 16), p["norm1_b"].reshape(1, 16),
        p["ff1_w"], p["ff1_b"].reshape(1, 2048),
        p["ff2_w"], p["ff2_b"].reshape(1, 16),
        p["norm2_g"].reshape(1, 16), p["norm2_b"].reshape(1, 16),
        p["enc_norm_g"].reshape(1, 16), p["enc_norm_b"].reshape(1, 16),
        p["pred_w"], p["pred_b"].reshape(1, 8),
        p["out0_w"], p["out0_b"].reshape(1, 4),
        p["out1_w"], p["out1_b"].reshape(1, 1),
    ]
    r1p, r2p = pl.pallas_call(
        _fused_body,
        out_shape=[
            jax.ShapeDtypeStruct((_R, 8), jnp.float32),
            jax.ShapeDtypeStruct((_R, 1), jnp.float32),
        ],
    )(*args)
    r1 = r1p.reshape(_S, _NP, 8)[:, :_N, :]
    r2 = r2p.reshape(_S, _NP, 1)[:, :_N, :]
    return (r1, r2, r2[-1])
